# baseline (speedup 1.0000x reference)
import functools

import jax
import jax.numpy as jnp
from jax import lax
from jax.experimental import pallas as pl
from jax.experimental.pallas import tpu as pltpu
from jax.experimental.pallas import tpu_sc as plsc

NC = 2
NS = 16
NW = NC * NS
CH = 128


def _make_segsum(n_chunks: int, n_acc: int, d: int):
  per_w = n_chunks // NW
  assert n_chunks % NW == 0
  rpt = n_acc // NS

  mesh = plsc.VectorSubcoreMesh(core_axis_name="c", subcore_axis_name="s")

  @functools.partial(
      pl.kernel,
      out_type=jax.ShapeDtypeStruct((NC, n_acc, d), jnp.float32),
      mesh=mesh,
      scratch_types=[
          pltpu.VMEM((CH,), jnp.int32),
          pltpu.VMEM((CH,), jnp.int32),
          pltpu.VMEM((CH, d), jnp.float32),
          pltpu.VMEM_SHARED((n_acc, d), jnp.float32),
          pltpu.SemaphoreType.DMA,
      ],
  )
  def segsum(table_hbm, src_hbm, dst_hbm, z_hbm, out_hbm,
             src_v, dst_v, rows_v, acc_sh, sem):
    cid = lax.axis_index("c")
    sid = lax.axis_index("s")
    wid = sid * NC + cid

    row0 = sid * rpt
    pltpu.sync_copy(z_hbm.at[pl.ds(row0, rpt)], acc_sh.at[pl.ds(row0, rpt)])
    plsc.subcore_barrier()

    def body(j, carry):
      base = (wid + j * NW) * CH
      pltpu.sync_copy(src_hbm.at[pl.ds(base, CH)], src_v)
      pltpu.sync_copy(dst_hbm.at[pl.ds(base, CH)], dst_v)
      pltpu.async_copy(table_hbm.at[src_v], rows_v, sem).wait()
      pltpu.sync_copy(rows_v, acc_sh.at[dst_v], add=True)
      return carry

    lax.fori_loop(0, per_w, body, 0)
    plsc.subcore_barrier()

    pltpu.sync_copy(acc_sh.at[pl.ds(row0, rpt)],
                    out_hbm.at[cid, pl.ds(row0, rpt)])

  return segsum


def _make_count(n_chunks: int, n_acc: int):
  per_w = n_chunks // NW
  assert n_chunks % NW == 0
  rpt = n_acc // NS
  w = 128

  mesh = plsc.VectorSubcoreMesh(core_axis_name="c", subcore_axis_name="s")

  @functools.partial(
      pl.kernel,
      out_type=jax.ShapeDtypeStruct((NC, n_acc, w), jnp.float32),
      mesh=mesh,
      scratch_types=[
          pltpu.VMEM((per_w, CH), jnp.int32),
          pltpu.VMEM((CH, w), jnp.float32),
          pltpu.VMEM_SHARED((n_acc, w), jnp.float32),
      ],
  )
  def count(dst_hbm, ones_hbm, z_hbm, out_hbm, dst_v, ones_v, acc_sh):
    cid = lax.axis_index("c")
    sid = lax.axis_index("s")
    wid = sid * NC + cid

    row0 = sid * rpt
    pltpu.sync_copy(z_hbm.at[pl.ds(row0, rpt)], acc_sh.at[pl.ds(row0, rpt)])
    pltpu.sync_copy(dst_hbm.at[pl.ds(wid * per_w, per_w)], dst_v)
    pltpu.sync_copy(ones_hbm, ones_v)
    plsc.subcore_barrier()

    def body(j, carry):
      pltpu.sync_copy(ones_v, acc_sh.at[dst_v.at[j]], add=True)
      return carry

    lax.fori_loop(0, per_w, body, 0)
    plsc.subcore_barrier()
    pltpu.sync_copy(acc_sh.at[pl.ds(row0, rpt)],
                    out_hbm.at[cid, pl.ds(row0, rpt)])

  return count


def _layer_body(h_ref, p0_ref, p1_ref, invd_ref, ws_ref, wn_ref, b_ref, o_ref):
  mean = (p0_ref[...] + p1_ref[...]) * invd_ref[...]
  acc = lax.dot_general(h_ref[...], ws_ref[...], (((1,), (0,)), ((), ())),
                        precision=lax.Precision.HIGHEST,
                        preferred_element_type=jnp.float32)
  acc = acc + lax.dot_general(mean, wn_ref[...], (((1,), (0,)), ((), ())),
                              precision=lax.Precision.HIGHEST,
                              preferred_element_type=jnp.float32)
  o_ref[...] = jnp.maximum(acc + b_ref[...], 0.0)


def _layer_tc(h, p0, p1, invd, w_self, w_neigh, b):
  n, d = h.shape
  blk = 2000
  bs_row = pl.BlockSpec((blk, d), lambda i: (i, 0))
  bs_w = pl.BlockSpec((d, d), lambda i: (0, 0))
  bs_b = pl.BlockSpec((1, d), lambda i: (0, 0))
  return pl.pallas_call(
      _layer_body,
      grid=(n // blk,),
      in_specs=[bs_row, bs_row, bs_row, bs_row, bs_w, bs_w, bs_b],
      out_specs=bs_row,
      out_shape=jax.ShapeDtypeStruct((n, d), jnp.float32),
  )(h, p0, p1, invd, w_self, w_neigh, b.reshape(1, d))


def _final_body(p0_ref, p1_ref, invc_ref, w_ref, b_ref, o_ref):
  pooled = (p0_ref[...] + p1_ref[...]) * invc_ref[...]
  o_ref[...] = lax.dot_general(pooled, w_ref[...], (((1,), (0,)), ((), ())),
                               precision=lax.Precision.HIGHEST,
                               preferred_element_type=jnp.float32) + b_ref[...]


def _final_tc(p0, p1, invc, lin_w, lin_b):
  g, d = p0.shape
  c = lin_w.shape[1]
  return pl.pallas_call(
      _final_body,
      out_shape=jax.ShapeDtypeStruct((g, c), jnp.float32),
  )(p0, p1, invc, lin_w, lin_b.reshape(1, c))


def kernel(x, edge_index, edge_attr, batch,
           W_self0, W_neigh0, b0,
           W_self1, W_neigh1, b1,
           W_self2, W_neigh2, b2,
           lin_W, lin_b):
  n, d = x.shape
  e = edge_index.shape[1]
  g = 64
  src = edge_index[0]
  dst = edge_index[1]

  n_acc_n = ((n + NS * 8 - 1) // (NS * 8)) * (NS * 8)

  epad = (-e) % (2 * NW * CH)
  src_e = jnp.concatenate([src, jnp.zeros((epad,), jnp.int32)])
  dst_e = jnp.concatenate([dst, jnp.full((epad,), n_acc_n - 1, jnp.int32)])
  n_ch = (e + epad) // CH
  src2 = src_e.reshape(n_ch, CH)
  dst2 = dst_e.reshape(n_ch, CH)

  count_edges = _make_count(n_ch, n_acc_n)
  ones128 = jnp.ones((CH, d), jnp.float32)
  zn = jnp.zeros((n_acc_n, d), jnp.float32)
  degp = count_edges(dst2, ones128, zn)
  deg = degp[0, :n, 0] + degp[1, :n, 0]
  invd = jnp.broadcast_to((1.0 / jnp.maximum(deg, 1.0))[:, None], (n, d))

  segsum_edges = _make_segsum(n_ch, n_acc_n, d)
  h = x
  prev = degp
  for w_self, w_neigh, b in ((W_self0, W_neigh0, b0),
                             (W_self1, W_neigh1, b1),
                             (W_self2, W_neigh2, b2)):
    zdep, _ = lax.optimization_barrier((zn, prev))
    parts = segsum_edges(h, src_e, dst_e, zdep)
    prev = parts
    h = _layer_tc(h, parts[0, :n], parts[1, :n], invd, w_self, w_neigh, b)

  n_acc = NS * 8
  ppad = (-n) % (2 * NW * CH)
  src_pp = jnp.concatenate([jnp.arange(n, dtype=jnp.int32),
                            jnp.zeros((ppad,), jnp.int32)])
  dst_pp = jnp.concatenate([batch, jnp.full((ppad,), n_acc - 1, jnp.int32)])
  n_chp = (n + ppad) // CH
  srcp2 = src_pp.reshape(n_chp, CH)
  dstp2 = dst_pp.reshape(n_chp, CH)

  segsum_pool = _make_segsum(n_chp, n_acc, d)
  zp = jnp.zeros((n_acc, d), jnp.float32)
  zpdep, _ = lax.optimization_barrier((zp, prev))
  pool_parts = segsum_pool(h, src_pp, dst_pp, zpdep)

  count_pool = _make_count(n_chp, n_acc)
  zpdep2, _ = lax.optimization_barrier((zp, pool_parts))
  cntp = count_pool(dstp2, ones128, zpdep2)
  cnt = cntp[0, :g, 0] + cntp[1, :g, 0]
  invc = jnp.broadcast_to((1.0 / jnp.maximum(cnt, 1.0))[:, None], (g, d))

  return _final_tc(pool_parts[0, :g], pool_parts[1, :g], invc, lin_W, lin_b)


# speedup vs baseline: 1.0027x; 1.0027x over previous
import functools

import jax
import jax.numpy as jnp
from jax import lax
from jax.experimental import pallas as pl
from jax.experimental.pallas import tpu as pltpu
from jax.experimental.pallas import tpu_sc as plsc

NC = 2
NS = 16
NW = NC * NS
CH = 128


def _make_segsum(n_chunks: int, n_acc: int, d: int):
  per_w = n_chunks // NW
  assert n_chunks % NW == 0
  rpt = n_acc // NS

  mesh = plsc.VectorSubcoreMesh(core_axis_name="c", subcore_axis_name="s")

  @functools.partial(
      pl.kernel,
      out_type=jax.ShapeDtypeStruct((NC, n_acc, d), jnp.float32),
      mesh=mesh,
      scratch_types=[
          pltpu.VMEM((CH,), jnp.int32),
          pltpu.VMEM((CH,), jnp.int32),
          pltpu.VMEM((CH, d), jnp.float32),
          pltpu.VMEM_SHARED((n_acc, d), jnp.float32),
          pltpu.SemaphoreType.DMA,
      ],
  )
  def segsum(table_hbm, src_hbm, dst_hbm, z_hbm, out_hbm,
             src_v, dst_v, rows_v, acc_sh, sem):
    cid = lax.axis_index("c")
    sid = lax.axis_index("s")
    wid = sid * NC + cid

    row0 = sid * rpt
    pltpu.sync_copy(z_hbm.at[pl.ds(row0, rpt)], acc_sh.at[pl.ds(row0, rpt)])
    plsc.subcore_barrier()

    def body(j, carry):
      base = (wid + j * NW) * CH
      pltpu.sync_copy(src_hbm.at[pl.ds(base, CH)], src_v)
      pltpu.sync_copy(dst_hbm.at[pl.ds(base, CH)], dst_v)
      pltpu.async_copy(table_hbm.at[src_v], rows_v, sem).wait()
      pltpu.sync_copy(rows_v, acc_sh.at[dst_v], add=True)
      return carry

    lax.fori_loop(0, per_w, body, 0)
    plsc.subcore_barrier()

    pltpu.sync_copy(acc_sh.at[pl.ds(row0, rpt)],
                    out_hbm.at[cid, pl.ds(row0, rpt)])

  return segsum


def _make_count(n_chunks: int, n_acc: int):
  per_w = n_chunks // NW
  assert n_chunks % NW == 0
  rpt = n_acc // NS
  w = 128

  mesh = plsc.VectorSubcoreMesh(core_axis_name="c", subcore_axis_name="s")

  @functools.partial(
      pl.kernel,
      out_type=jax.ShapeDtypeStruct((NC, n_acc, w), jnp.float32),
      mesh=mesh,
      scratch_types=[
          pltpu.VMEM((per_w, CH), jnp.int32),
          pltpu.VMEM((CH, w), jnp.float32),
          pltpu.VMEM_SHARED((n_acc, w), jnp.float32),
      ],
  )
  def count(dst_hbm, ones_hbm, z_hbm, out_hbm, dst_v, ones_v, acc_sh):
    cid = lax.axis_index("c")
    sid = lax.axis_index("s")
    wid = sid * NC + cid

    row0 = sid * rpt
    pltpu.sync_copy(z_hbm.at[pl.ds(row0, rpt)], acc_sh.at[pl.ds(row0, rpt)])
    pltpu.sync_copy(dst_hbm.at[pl.ds(wid * per_w, per_w)], dst_v)
    pltpu.sync_copy(ones_hbm, ones_v)
    plsc.subcore_barrier()

    def body(j, carry):
      pltpu.sync_copy(ones_v, acc_sh.at[dst_v.at[j]], add=True)
      return carry

    lax.fori_loop(0, per_w, body, 0)
    plsc.subcore_barrier()
    pltpu.sync_copy(acc_sh.at[pl.ds(row0, rpt)],
                    out_hbm.at[cid, pl.ds(row0, rpt)])

  return count


def _layer_body(h_ref, p0_ref, p1_ref, invd_ref, ws_ref, wn_ref, b_ref, o_ref):
  mean = (p0_ref[...] + p1_ref[...]) * invd_ref[...]
  acc = lax.dot_general(h_ref[...], ws_ref[...], (((1,), (0,)), ((), ())),
                        precision=lax.Precision.HIGHEST,
                        preferred_element_type=jnp.float32)
  acc = acc + lax.dot_general(mean, wn_ref[...], (((1,), (0,)), ((), ())),
                              precision=lax.Precision.HIGHEST,
                              preferred_element_type=jnp.float32)
  o_ref[...] = jnp.maximum(acc + b_ref[...], 0.0)


def _layer_tc(h, p0, p1, invd, w_self, w_neigh, b):
  n, d = h.shape
  blk = 2000
  bs_row = pl.BlockSpec((blk, d), lambda i: (i, 0))
  bs_w = pl.BlockSpec((d, d), lambda i: (0, 0))
  bs_b = pl.BlockSpec((1, d), lambda i: (0, 0))
  return pl.pallas_call(
      _layer_body,
      grid=(n // blk,),
      in_specs=[bs_row, bs_row, bs_row, bs_row, bs_w, bs_w, bs_b],
      out_specs=bs_row,
      out_shape=jax.ShapeDtypeStruct((n, d), jnp.float32),
  )(h, p0, p1, invd, w_self, w_neigh, b.reshape(1, d))


def _final_body(p0_ref, p1_ref, invc_ref, w_ref, b_ref, o_ref):
  pooled = (p0_ref[...] + p1_ref[...]) * invc_ref[...]
  o_ref[...] = lax.dot_general(pooled, w_ref[...], (((1,), (0,)), ((), ())),
                               precision=lax.Precision.HIGHEST,
                               preferred_element_type=jnp.float32) + b_ref[...]


def _final_tc(p0, p1, invc, lin_w, lin_b):
  g, d = p0.shape
  c = lin_w.shape[1]
  return pl.pallas_call(
      _final_body,
      out_shape=jax.ShapeDtypeStruct((g, c), jnp.float32),
  )(p0, p1, invc, lin_w, lin_b.reshape(1, c))


def kernel(x, edge_index, edge_attr, batch,
           W_self0, W_neigh0, b0,
           W_self1, W_neigh1, b1,
           W_self2, W_neigh2, b2,
           lin_W, lin_b):
  n, d = x.shape
  e = edge_index.shape[1]
  g = 64
  src = edge_index[0]
  dst = edge_index[1]

  n_acc_n = ((n + NS * 8 - 1) // (NS * 8)) * (NS * 8)

  epad = (-e) % (2 * NW * CH)
  pad_rows = n + (jnp.arange(epad, dtype=jnp.int32) % (n_acc_n - n))
  src_e = jnp.concatenate([src, jnp.zeros((epad,), jnp.int32)])
  dst_e = jnp.concatenate([dst, pad_rows])
  n_ch = (e + epad) // CH
  src2 = src_e.reshape(n_ch, CH)
  dst2 = dst_e.reshape(n_ch, CH)

  count_edges = _make_count(n_ch, n_acc_n)
  ones128 = jnp.ones((CH, d), jnp.float32)
  zn = jnp.zeros((n_acc_n, d), jnp.float32)
  degp = count_edges(dst2, ones128, zn)
  deg = degp[0, :n, 0] + degp[1, :n, 0]
  invd = jnp.broadcast_to((1.0 / jnp.maximum(deg, 1.0))[:, None], (n, d))

  segsum_edges = _make_segsum(n_ch, n_acc_n, d)
  h = x
  prev = degp
  for w_self, w_neigh, b in ((W_self0, W_neigh0, b0),
                             (W_self1, W_neigh1, b1),
                             (W_self2, W_neigh2, b2)):
    zdep, _ = lax.optimization_barrier((zn, prev))
    parts = segsum_edges(h, src_e, dst_e, zdep)
    prev = parts
    h = _layer_tc(h, parts[0, :n], parts[1, :n], invd, w_self, w_neigh, b)

  n_acc = NS * 8
  ppad = (-n) % (2 * NW * CH)
  src_pp = jnp.concatenate([jnp.arange(n, dtype=jnp.int32),
                            jnp.zeros((ppad,), jnp.int32)])
  pool_pad_rows = g + (jnp.arange(ppad, dtype=jnp.int32) % (n_acc - g))
  dst_pp = jnp.concatenate([batch, pool_pad_rows])
  n_chp = (n + ppad) // CH
  srcp2 = src_pp.reshape(n_chp, CH)
  dstp2 = dst_pp.reshape(n_chp, CH)

  segsum_pool = _make_segsum(n_chp, n_acc, d)
  zp = jnp.zeros((n_acc, d), jnp.float32)
  zpdep, _ = lax.optimization_barrier((zp, prev))
  pool_parts = segsum_pool(h, src_pp, dst_pp, zpdep)

  count_pool = _make_count(n_chp, n_acc)
  zpdep2, _ = lax.optimization_barrier((zp, pool_parts))
  cntp = count_pool(dstp2, ones128, zpdep2)
  cnt = cntp[0, :g, 0] + cntp[1, :g, 0]
  invc = jnp.broadcast_to((1.0 / jnp.maximum(cnt, 1.0))[:, None], (g, d))

  return _final_tc(pool_parts[0, :g], pool_parts[1, :g], invc, lin_W, lin_b)


# speedup vs baseline: 2.1202x; 2.1146x over previous
import functools

import jax
import jax.numpy as jnp
from jax import lax
from jax.experimental import pallas as pl
from jax.experimental.pallas import tpu as pltpu
from jax.experimental.pallas import tpu_sc as plsc

NC = 2
NS = 16
NW = NC * NS
CH = 128


def _make_segsum(n_chunks: int, n_acc: int, d: int):
  base_ch, rem_ch = divmod(n_chunks, NW)
  rpt = n_acc // NS

  mesh = plsc.VectorSubcoreMesh(core_axis_name="c", subcore_axis_name="s")

  @functools.partial(
      pl.kernel,
      out_type=jax.ShapeDtypeStruct((NC, n_acc, d), jnp.float32),
      mesh=mesh,
      scratch_types=[
          pltpu.VMEM((CH,), jnp.int32),
          pltpu.VMEM((CH,), jnp.int32),
          pltpu.VMEM((CH, d), jnp.float32),
          pltpu.VMEM_SHARED((n_acc, d), jnp.float32),
          pltpu.SemaphoreType.DMA,
      ],
  )
  def segsum(table_hbm, src_hbm, dst_hbm, z_hbm, out_hbm,
             src_v, dst_v, rows_v, acc_sh, sem):
    cid = lax.axis_index("c")
    sid = lax.axis_index("s")
    wid = sid * NC + cid

    row0 = sid * rpt
    pltpu.sync_copy(z_hbm.at[pl.ds(row0, rpt)], acc_sh.at[pl.ds(row0, rpt)])
    plsc.subcore_barrier()

    n_mine = base_ch + jnp.where(wid < rem_ch, 1, 0)

    def body(j, carry):
      base = (wid + j * NW) * CH
      pltpu.sync_copy(src_hbm.at[pl.ds(base, CH)], src_v)
      pltpu.sync_copy(dst_hbm.at[pl.ds(base, CH)], dst_v)
      pltpu.async_copy(table_hbm.at[src_v], rows_v, sem).wait()
      pltpu.sync_copy(rows_v, acc_sh.at[dst_v], add=True)
      return carry

    lax.fori_loop(0, n_mine, body, 0)
    plsc.subcore_barrier()

    pltpu.sync_copy(acc_sh.at[pl.ds(row0, rpt)],
                    out_hbm.at[cid, pl.ds(row0, rpt)])

  return segsum


def _make_count(n_chunks: int, n_acc: int):
  per_w = n_chunks // NW
  assert n_chunks % NW == 0
  rpt = n_acc // NS
  w = 128

  mesh = plsc.VectorSubcoreMesh(core_axis_name="c", subcore_axis_name="s")

  @functools.partial(
      pl.kernel,
      out_type=jax.ShapeDtypeStruct((NC, n_acc, w), jnp.float32),
      mesh=mesh,
      scratch_types=[
          pltpu.VMEM((per_w, CH), jnp.int32),
          pltpu.VMEM((CH, w), jnp.float32),
          pltpu.VMEM_SHARED((n_acc, w), jnp.float32),
      ],
  )
  def count(dst_hbm, ones_hbm, z_hbm, out_hbm, dst_v, ones_v, acc_sh):
    cid = lax.axis_index("c")
    sid = lax.axis_index("s")
    wid = sid * NC + cid

    row0 = sid * rpt
    pltpu.sync_copy(z_hbm.at[pl.ds(row0, rpt)], acc_sh.at[pl.ds(row0, rpt)])
    pltpu.sync_copy(dst_hbm.at[pl.ds(wid * per_w, per_w)], dst_v)
    pltpu.sync_copy(ones_hbm, ones_v)
    plsc.subcore_barrier()

    def body(j, carry):
      pltpu.sync_copy(ones_v, acc_sh.at[dst_v.at[j]], add=True)
      return carry

    lax.fori_loop(0, per_w, body, 0)
    plsc.subcore_barrier()
    pltpu.sync_copy(acc_sh.at[pl.ds(row0, rpt)],
                    out_hbm.at[cid, pl.ds(row0, rpt)])

  return count


def _layer_body(h_ref, p0_ref, p1_ref, invd_ref, ws_ref, wn_ref, b_ref, o_ref):
  mean = (p0_ref[...] + p1_ref[...]) * invd_ref[...]
  acc = lax.dot_general(h_ref[...], ws_ref[...], (((1,), (0,)), ((), ())),
                        precision=lax.Precision.HIGHEST,
                        preferred_element_type=jnp.float32)
  acc = acc + lax.dot_general(mean, wn_ref[...], (((1,), (0,)), ((), ())),
                              precision=lax.Precision.HIGHEST,
                              preferred_element_type=jnp.float32)
  o_ref[...] = jnp.maximum(acc + b_ref[...], 0.0)


def _layer_tc(h, p0, p1, invd, w_self, w_neigh, b):
  n, d = h.shape
  blk = 2000
  bs_row = pl.BlockSpec((blk, d), lambda i: (i, 0))
  bs_w = pl.BlockSpec((d, d), lambda i: (0, 0))
  bs_b = pl.BlockSpec((1, d), lambda i: (0, 0))
  return pl.pallas_call(
      _layer_body,
      grid=(n // blk,),
      in_specs=[bs_row, bs_row, bs_row, bs_row, bs_w, bs_w, bs_b],
      out_specs=bs_row,
      out_shape=jax.ShapeDtypeStruct((n, d), jnp.float32),
  )(h, p0, p1, invd, w_self, w_neigh, b.reshape(1, d))


def _final_body(p0_ref, p1_ref, invc_ref, w_ref, b_ref, o_ref):
  pooled = (p0_ref[...] + p1_ref[...]) * invc_ref[...]
  o_ref[...] = lax.dot_general(pooled, w_ref[...], (((1,), (0,)), ((), ())),
                               precision=lax.Precision.HIGHEST,
                               preferred_element_type=jnp.float32) + b_ref[...]


def _final_tc(p0, p1, invc, lin_w, lin_b):
  g, d = p0.shape
  c = lin_w.shape[1]
  return pl.pallas_call(
      _final_body,
      out_shape=jax.ShapeDtypeStruct((g, c), jnp.float32),
  )(p0, p1, invc, lin_w, lin_b.reshape(1, c))


def kernel(x, edge_index, edge_attr, batch,
           W_self0, W_neigh0, b0,
           W_self1, W_neigh1, b1,
           W_self2, W_neigh2, b2,
           lin_W, lin_b):
  n, d = x.shape
  e = edge_index.shape[1]
  g = 64
  src = edge_index[0]
  dst = edge_index[1]

  n_acc_n = ((n + NS * 8 - 1) // (NS * 8)) * (NS * 8)

  epad = (-e) % (8 * NW * CH)
  pad_rows = n + (jnp.arange(epad, dtype=jnp.int32) % (n_acc_n - n))
  dst2 = jnp.concatenate([dst, pad_rows]).reshape((e + epad) // CH, CH)

  count_edges = _make_count((e + epad) // CH, n_acc_n)
  ones128 = jnp.ones((CH, d), jnp.float32)
  zn = jnp.zeros((n_acc_n, d), jnp.float32)
  degp = count_edges(dst2, ones128, zn)
  deg = degp[0, :n, 0] + degp[1, :n, 0]
  invd = jnp.broadcast_to((1.0 / jnp.maximum(deg, 1.0))[:, None], (n, d))

  segsum_edges = _make_segsum(e // CH, n_acc_n, d)
  h = x
  prev = degp
  for w_self, w_neigh, b in ((W_self0, W_neigh0, b0),
                             (W_self1, W_neigh1, b1),
                             (W_self2, W_neigh2, b2)):
    zdep, _ = lax.optimization_barrier((zn, prev))
    parts = segsum_edges(h, src, dst, zdep)
    prev = parts
    h = _layer_tc(h, parts[0, :n], parts[1, :n], invd, w_self, w_neigh, b)

  n_acc = NS * 8
  ppad = (-n) % CH
  src_pp = jnp.concatenate([jnp.arange(n, dtype=jnp.int32),
                            jnp.zeros((ppad,), jnp.int32)])
  pool_pad_rows = g + (jnp.arange(ppad, dtype=jnp.int32) % (n_acc - g))
  dst_pp = jnp.concatenate([batch, pool_pad_rows])
  n_chp = (n + ppad) // CH

  segsum_pool = _make_segsum(n_chp, n_acc, d)
  zp = jnp.zeros((n_acc, d), jnp.float32)
  zpdep, _ = lax.optimization_barrier((zp, prev))
  pool_parts = segsum_pool(h, src_pp, dst_pp, zpdep)

  cpad = (-(n + ppad)) % (8 * NW * CH)
  dst_pc = jnp.concatenate(
      [dst_pp, g + (jnp.arange(cpad, dtype=jnp.int32) % (n_acc - g))])
  n_chc = (n + ppad + cpad) // CH
  count_pool = _make_count(n_chc, n_acc)
  zpdep2, _ = lax.optimization_barrier((zp, pool_parts))
  cntp = count_pool(dst_pc.reshape(n_chc, CH), ones128, zpdep2)
  cnt = cntp[0, :g, 0] + cntp[1, :g, 0]
  invc = jnp.broadcast_to((1.0 / jnp.maximum(cnt, 1.0))[:, None], (g, d))

  return _final_tc(pool_parts[0, :g], pool_parts[1, :g], invc, lin_W, lin_b)
